# baseline (speedup 1.0000x reference)
import functools
import jax
import jax.numpy as jnp
from jax import lax
from jax.experimental import pallas as pl
from jax.experimental.pallas import tpu as pltpu
from jax.experimental.pallas import tpu_sc as plsc

_N = 62
_F = 5
_H = 64
_C = 3
_E = _N * _N
_EP = 3856
_SP = 3968
_NF = _N * _F
_NH = _N * _H
_NTRIL = _N * (_N + 1) // 2
_NTRILP = 1968


def _sc_edge_body(rows_hbm, cols_hbm, wp_hbm, cflat_hbm, deg_hbm,
                  rows_v, cols_v, wp_v, deg_v, zero_v,
                  didx_v, dval_v, cidx_v, cval_v, deg_sh, c_sh):
    wid = lax.axis_index("s") * 2 + lax.axis_index("c")

    @pl.when(wid == 0)
    def _():
        pltpu.sync_copy(rows_hbm, rows_v)
        pltpu.sync_copy(cols_hbm, cols_v)
        pltpu.sync_copy(wp_hbm, wp_v)
        it = lax.iota(jnp.int32, 16)
        z16 = jnp.zeros((16,), jnp.float32)
        for k in range(4):
            deg_v[pl.ds(k * 16, 16)] = z16
        for k in range(_SP // 16):
            zero_v[pl.ds(k * 16, 16)] = z16

        def p1(i, carry):
            base = i * 16
            r = rows_v[pl.ds(base, 16)]
            c = cols_v[pl.ds(base, 16)]
            e = base + it
            ii = e // _N
            jj = e - ii * _N
            aa = jnp.maximum(ii, jj)
            bb = jnp.minimum(ii, jj)
            t = ((aa * (aa + 1)) >> 1) + bb
            w = plsc.load_gather(wp_v, [t])
            is_self = r == c
            valid = e < _E
            w_ns = jnp.where(is_self | ~valid, 0.0, w)
            row2 = i // 8
            off2 = (i - row2 * 8) * 16
            didx_v[row2, pl.ds(off2, 16)] = r
            dval_v[row2, pl.ds(off2, 16)] = jnp.abs(w_ns)
            cidx_v[row2, pl.ds(off2, 16)] = c * _N + r
            cval_v[row2, pl.ds(off2, 16)] = w_ns
            return carry

        lax.fori_loop(0, 241, p1, 0)
        for k in range(7):
            off = 16 + k * 16
            didx_v[30, pl.ds(off, 16)] = jnp.zeros((16,), jnp.int32)
            dval_v[30, pl.ds(off, 16)] = z16
            cidx_v[30, pl.ds(off, 16)] = jnp.zeros((16,), jnp.int32)
            cval_v[30, pl.ds(off, 16)] = z16

        pltpu.sync_copy(deg_v, deg_sh)
        pltpu.sync_copy(zero_v, c_sh)
        for j in range(31):
            pltpu.sync_copy(dval_v.at[j], deg_sh.at[didx_v.at[j]], add=True)
            pltpu.sync_copy(cval_v.at[j], c_sh.at[cidx_v.at[j]], add=True)

        pltpu.sync_copy(c_sh, cflat_hbm)
        pltpu.sync_copy(deg_sh, deg_v)
        pltpu.sync_copy(deg_v, deg_hbm)


_sc_edge = functools.partial(
    pl.kernel,
    out_type=(jax.ShapeDtypeStruct((_SP,), jnp.float32),
              jax.ShapeDtypeStruct((64,), jnp.float32)),
    mesh=plsc.VectorSubcoreMesh(core_axis_name="c", subcore_axis_name="s"),
    scratch_types=[
        pltpu.VMEM((_EP,), jnp.int32),
        pltpu.VMEM((_EP,), jnp.int32),
        pltpu.VMEM((_NTRILP,), jnp.float32),
        pltpu.VMEM((64,), jnp.float32),
        pltpu.VMEM((_SP,), jnp.float32),
        pltpu.VMEM((31, 128), jnp.int32),
        pltpu.VMEM((31, 128), jnp.float32),
        pltpu.VMEM((31, 128), jnp.int32),
        pltpu.VMEM((31, 128), jnp.float32),
        pltpu.VMEM_SHARED((64,), jnp.float32),
        pltpu.VMEM_SHARED((_SP,), jnp.float32),
    ],
    compiler_params=pltpu.CompilerParams(needs_layout_passes=False),
)(_sc_edge_body)


def _tc_kernel(ei_ref, wp_ref, cm_ref, degp_ref, degl_ref, w1_ref, b1_ref,
               w2_ref, b2_ref, x_ref, out_ref, a12_ref):
    @pl.when(pl.program_id(0) == 0)
    def _build():
        r = ei_ref[0:1, :]
        c = ei_ref[1:2, :]
        zrow = jnp.zeros((1, _N), jnp.float32)
        rows = []
        for i in range(_N):
            lo = i * (i + 1) // 2
            sl = wp_ref[0:1, lo:lo + i + 1]
            rows.append(sl if i == _N - 1
                        else jnp.concatenate([sl, zrow[:, :_N - 1 - i]],
                                             axis=1))
        tril = jnp.concatenate(rows, axis=0)
        eye = (lax.broadcasted_iota(jnp.int32, (_N, _N), 0)
               == lax.broadcasted_iota(jnp.int32, (_N, _N), 1)
               ).astype(jnp.float32)
        wm = tril + tril.T - tril * eye
        w = jnp.concatenate([wm[i:i + 1, :] for i in range(_N)], axis=1)

        nodes = lax.broadcasted_iota(jnp.int32, (_N, _E), 0)
        oh_r = r == nodes
        is_self = r == c
        e_iota = lax.broadcasted_iota(jnp.int32, (_N, _E), 1)
        self_at = oh_r & is_self
        e_sel = jnp.max(jnp.where(self_at, e_iota, -1), axis=1, keepdims=True)
        has_self = e_sel >= 0
        win = (e_iota == e_sel) & self_at
        self_val = jnp.sum(jnp.where(win, w, 0.0), axis=1, keepdims=True)
        loop_w = jnp.where(has_self, self_val, 1.0)

        deg_col = degp_ref[0:_N, 0:1] + jnp.abs(loop_w)
        dis_col = jnp.where(deg_col == 0.0, 0.0, lax.rsqrt(deg_col))
        deg_row = degl_ref[0:1, 0:_N] + jnp.abs(loop_w).T
        dis_row = jnp.where(deg_row == 0.0, 0.0, lax.rsqrt(deg_row))

        m = dis_col * cm_ref[...] * dis_row
        m = m + eye * (dis_col * dis_col * loop_w)
        m2 = jnp.dot(m, m, preferred_element_type=jnp.float32)

        sel_j = (lax.broadcasted_iota(jnp.int32, (_NF, _N), 0) // _F
                 == lax.broadcasted_iota(jnp.int32, (_NF, _N), 1)
                 ).astype(jnp.float32)
        sel_i = (lax.broadcasted_iota(jnp.int32, (_N, _NH), 0)
                 == lax.broadcasted_iota(jnp.int32, (_N, _NH), 1) // _H
                 ).astype(jnp.float32)
        d1 = jnp.dot(jnp.dot(sel_j, m2.T, preferred_element_type=jnp.float32),
                     sel_i, preferred_element_type=jnp.float32)
        sel_f = (lax.broadcasted_iota(jnp.int32, (_NF, _F), 0) % _F
                 == lax.broadcasted_iota(jnp.int32, (_NF, _F), 1)
                 ).astype(jnp.float32)
        sel_h = (lax.broadcasted_iota(jnp.int32, (_H, _NH), 0)
                 == lax.broadcasted_iota(jnp.int32, (_H, _NH), 1) % _H
                 ).astype(jnp.float32)
        d2 = jnp.dot(jnp.dot(sel_f, w1_ref[...],
                             preferred_element_type=jnp.float32),
                     sel_h, preferred_element_type=jnp.float32)
        a12_ref[...] = d1 * d2

    sel_h2 = (lax.broadcasted_iota(jnp.int32, (_NH, _H), 0) % _H
              == lax.broadcasted_iota(jnp.int32, (_NH, _H), 1)
              ).astype(jnp.float32)
    b1t = lax.dot_general(b1_ref[...], sel_h2, (((1,), (1,)), ((), ())),
                          preferred_element_type=jnp.float32)
    a3 = jnp.dot(sel_h2, w2_ref[...], preferred_element_type=jnp.float32)
    h = jnp.dot(x_ref[...], a12_ref[...], preferred_element_type=jnp.float32)
    h = jnp.maximum(h + b1t, 0.0)
    out_ref[...] = (jnp.dot(h, a3, preferred_element_type=jnp.float32)
                    + b2_ref[...])


def kernel(x, edge_index, edge_weight_param, W1, b1, W2, b2):
    B = x.shape[0]
    pad_e = _EP - _E
    rows_p = jnp.concatenate([edge_index[0], jnp.zeros((pad_e,), jnp.int32)])
    cols_p = jnp.concatenate([edge_index[1], jnp.ones((pad_e,), jnp.int32)])
    wp_p = jnp.concatenate([edge_weight_param,
                            jnp.zeros((_NTRILP - _NTRIL,), jnp.float32)])
    cflat, degp = _sc_edge(rows_p, cols_p, wp_p)
    cm = cflat[:_E].reshape(_N, _N)
    x2 = x.reshape(B, _NF)
    G = 1024
    out = pl.pallas_call(
        _tc_kernel,
        grid=(B // G,),
        in_specs=[
            pl.BlockSpec((2, _E), lambda i: (0, 0)),
            pl.BlockSpec((1, _NTRIL), lambda i: (0, 0)),
            pl.BlockSpec((_N, _N), lambda i: (0, 0)),
            pl.BlockSpec((64, 1), lambda i: (0, 0)),
            pl.BlockSpec((1, 64), lambda i: (0, 0)),
            pl.BlockSpec((_F, _H), lambda i: (0, 0)),
            pl.BlockSpec((1, _H), lambda i: (0, 0)),
            pl.BlockSpec((_H, _C), lambda i: (0, 0)),
            pl.BlockSpec((1, _C), lambda i: (0, 0)),
            pl.BlockSpec((G, _NF), lambda i: (i, 0)),
        ],
        out_specs=pl.BlockSpec((G, _C), lambda i: (i, 0)),
        out_shape=jax.ShapeDtypeStruct((B, _C), jnp.float32),
        scratch_shapes=[pltpu.VMEM((_NF, _NH), jnp.float32)],
    )(edge_index, edge_weight_param.reshape(1, _NTRIL), cm,
      degp.reshape(64, 1), degp.reshape(1, 64), W1, b1.reshape(1, _H),
      W2, b2.reshape(1, _C), x2)
    return out


# speedup vs baseline: 2.5593x; 2.5593x over previous
import jax
import jax.numpy as jnp
from jax.experimental import pallas as pl
from jax.experimental.pallas import tpu as pltpu

_N = 62
_F = 5
_H = 64
_C = 3
_E = _N * _N
_NF = _N * _F
_NH = _N * _H


def _fused_kernel(ei_ref, wp_ref, w1_ref, b1_ref, w2_ref, b2_ref, x_ref,
                  out_ref, a12_ref):
    @pl.when(pl.program_id(0) == 0)
    def _edge_phase():
        r = ei_ref[0:1, :]
        c = ei_ref[1:2, :]

        zrow = jnp.zeros((1, _N), jnp.float32)
        rows = []
        for i in range(_N):
            lo = i * (i + 1) // 2
            sl = wp_ref[0:1, lo:lo + i + 1]
            rows.append(sl if i == _N - 1
                        else jnp.concatenate([sl, zrow[:, :_N - 1 - i]], axis=1))
        tril = jnp.concatenate(rows, axis=0)
        eye = (jax.lax.broadcasted_iota(jnp.int32, (_N, _N), 0)
               == jax.lax.broadcasted_iota(jnp.int32, (_N, _N), 1)
               ).astype(jnp.float32)
        wm = tril + tril.T - tril * eye
        w = jnp.concatenate([wm[i:i + 1, :] for i in range(_N)], axis=1)

        nodes = jax.lax.broadcasted_iota(jnp.int32, (_N, _E), 0)
        oh_r = r == nodes
        oh_c = c == nodes
        ohr_f = oh_r.astype(jnp.float32)
        ohc_f = oh_c.astype(jnp.float32)
        is_self = r == c
        w_ns = jnp.where(is_self, 0.0, w)

        e_iota = jax.lax.broadcasted_iota(jnp.int32, (_N, _E), 1)
        self_at = oh_r & is_self
        e_sel = jnp.max(jnp.where(self_at, e_iota, -1), axis=1, keepdims=True)
        has_self = e_sel >= 0
        win = (e_iota == e_sel) & self_at
        self_val = jnp.sum(jnp.where(win, w, 0.0), axis=1, keepdims=True)
        loop_w = jnp.where(has_self, self_val, 1.0)

        deg = (jnp.sum(ohr_f * jnp.abs(w_ns), axis=1, keepdims=True)
               + jnp.abs(loop_w))
        dis = jnp.where(deg == 0.0, 0.0, jax.lax.rsqrt(deg))

        dis_r = jnp.sum(ohr_f * dis, axis=0, keepdims=True)
        dis_c = jnp.sum(ohc_f * dis, axis=0, keepdims=True)
        norm = dis_r * w_ns * dis_c

        m_msg = jax.lax.dot_general(
            ohc_f * norm, ohr_f, (((1,), (1,)), ((), ())),
            preferred_element_type=jnp.float32)
        m = m_msg + eye * (dis * dis * loop_w)
        m2 = jnp.dot(m, m, preferred_element_type=jnp.float32)

        sel_j = (jax.lax.broadcasted_iota(jnp.int32, (_NF, _N), 0) // _F
                 == jax.lax.broadcasted_iota(jnp.int32, (_NF, _N), 1)
                 ).astype(jnp.float32)
        sel_i = (jax.lax.broadcasted_iota(jnp.int32, (_N, _NH), 0)
                 == jax.lax.broadcasted_iota(jnp.int32, (_N, _NH), 1) // _H
                 ).astype(jnp.float32)
        d1 = jnp.dot(jnp.dot(sel_j, m2.T, preferred_element_type=jnp.float32),
                     sel_i, preferred_element_type=jnp.float32)
        sel_f = (jax.lax.broadcasted_iota(jnp.int32, (_NF, _F), 0) % _F
                 == jax.lax.broadcasted_iota(jnp.int32, (_NF, _F), 1)
                 ).astype(jnp.float32)
        sel_h = (jax.lax.broadcasted_iota(jnp.int32, (_H, _NH), 0)
                 == jax.lax.broadcasted_iota(jnp.int32, (_H, _NH), 1) % _H
                 ).astype(jnp.float32)
        d2 = jnp.dot(jnp.dot(sel_f, w1_ref[...],
                             preferred_element_type=jnp.float32),
                     sel_h, preferred_element_type=jnp.float32)
        a12_ref[...] = d1 * d2

    sel_h2 = (jax.lax.broadcasted_iota(jnp.int32, (_NH, _H), 0) % _H
              == jax.lax.broadcasted_iota(jnp.int32, (_NH, _H), 1)
              ).astype(jnp.float32)
    b1t = jax.lax.dot_general(b1_ref[...], sel_h2, (((1,), (1,)), ((), ())),
                              preferred_element_type=jnp.float32)
    a3 = jnp.dot(sel_h2, w2_ref[...],
                 preferred_element_type=jnp.float32)
    h = jnp.dot(x_ref[...], a12_ref[...], preferred_element_type=jnp.float32)
    h = jnp.maximum(h + b1t, 0.0)
    out_ref[...] = (jnp.dot(h, a3, preferred_element_type=jnp.float32)
                    + b2_ref[...])


def kernel(x, edge_index, edge_weight_param, W1, b1, W2, b2):
    B = x.shape[0]
    n_tril = _N * (_N + 1) // 2
    x2 = x.reshape(B, _NF)

    G = 1024
    out = pl.pallas_call(
        _fused_kernel,
        grid=(B // G,),
        in_specs=[
            pl.BlockSpec((2, _E), lambda i: (0, 0)),
            pl.BlockSpec((1, n_tril), lambda i: (0, 0)),
            pl.BlockSpec((_F, _H), lambda i: (0, 0)),
            pl.BlockSpec((1, _H), lambda i: (0, 0)),
            pl.BlockSpec((_H, _C), lambda i: (0, 0)),
            pl.BlockSpec((1, _C), lambda i: (0, 0)),
            pl.BlockSpec((G, _NF), lambda i: (i, 0)),
        ],
        out_specs=pl.BlockSpec((G, _C), lambda i: (i, 0)),
        out_shape=jax.ShapeDtypeStruct((B, _C), jnp.float32),
        scratch_shapes=[pltpu.VMEM((_NF, _NH), jnp.float32)],
    )(edge_index, edge_weight_param.reshape(1, n_tril), W1,
      b1.reshape(1, _H), W2, b2.reshape(1, _C), x2)
    return out
